# Initial kernel scaffold; baseline (speedup 1.0000x reference)
import jax
import jax.numpy as jnp
from jax.experimental import pallas as pl


def kernel(x, edge_index, edge_attr, batch, atom_tables, bond_tables, gcn_W, gcn_b, norm_gamma, norm_beta):
    raise NotImplementedError("write your pallas kernel here")


# speedup vs baseline: 1.7902x; 1.7902x over previous
import functools

import jax
import jax.numpy as jnp
from jax import lax
from jax.experimental import pallas as pl
from jax.experimental.pallas import tpu as pltpu
from jax.experimental.pallas import tpu_sc as plsc

N = 10000
E = 320000
H = 128
HH = 64
G = 64
EPS = 1e-7

NSUB = 16
CH = 64
NN = 10240
EP = 327680
NPT = NN // NSUB
NCHUNK_N = NPT // CH
EPT = EP // NSUB
NECHUNK = EPT // CH
NR = 10048


def _tc_prep_body(xT_ref, ei_ref, eaT_ref, at_ref, bd_ref,
                  h0_ref, fused_ref, ep_ref):
  h = jnp.zeros((N, H), jnp.float32)
  for f in range(3):
    xf = xT_ref[f].reshape(N, 1)
    onehot = (xf == lax.broadcasted_iota(jnp.int32, (N, 64), 1)
              ).astype(jnp.float32)
    h = h + lax.dot_general(onehot, at_ref[f], (((1,), (0,)), ((), ())),
                            preferred_element_type=jnp.float32)
  h0_ref[pl.ds(0, N), :] = h
  h0_ref[pl.ds(N, NN - N), :] = jnp.zeros((NN - N, H), jnp.float32)

  r = lax.broadcasted_iota(jnp.int32, (512, 1), 0)
  fu = jnp.zeros((512, H), jnp.float32)
  for f, sel in enumerate((r // 64, (r // 8) % 8, r % 8)):
    onehot = (sel == lax.broadcasted_iota(jnp.int32, (512, 8), 1)
              ).astype(jnp.float32)
    fu = fu + lax.dot_general(onehot, bd_ref[f], (((1,), (0,)), ((), ())),
                              preferred_element_type=jnp.float32)
  fused_ref[...] = fu

  ep_ref[pl.ds(0, E)] = ei_ref[0]
  ep_ref[pl.ds(E, EP - E)] = jnp.zeros((EP - E,), jnp.int32)
  ep_ref[pl.ds(EP, E)] = ei_ref[1]
  ep_ref[pl.ds(EP + E, EP - E)] = jnp.full((EP - E,), N, jnp.int32)
  ep_ref[pl.ds(2 * EP, E)] = eaT_ref[0] * 64 + eaT_ref[1] * 8 + eaT_ref[2]
  ep_ref[pl.ds(2 * EP + E, EP - E)] = jnp.zeros((EP - E,), jnp.int32)


def _tc_prep(xT, ei, eaT, at, bd):
  return pl.pallas_call(
      _tc_prep_body,
      out_shape=[
          jax.ShapeDtypeStruct((NN, H), jnp.float32),
          jax.ShapeDtypeStruct((512, H), jnp.float32),
          jax.ShapeDtypeStruct((3 * EP,), jnp.int32),
      ],
  )(xT, ei, eaT, at, bd)


def _sc_body(hin, fused, ep,
             agg_out,
             acc_sp,
             idx_src, idx_dst, idx_f,
             hsrc_v, emb_v, exc_v, out_v, sem):
  c = lax.axis_index("c")
  s = lax.axis_index("s")
  cb = c * HH
  nbase = s * NPT
  nchunks = jnp.where(s == NSUB - 1, NCHUNK_N - 3, NCHUNK_N)

  def zero_row(i, carry):
    for k in range(H // 16):
      sl = pl.ds(16 * k, 16)
      exc_v[i, sl] = jnp.zeros((16,), jnp.float32)
    return carry
  lax.fori_loop(0, CH, zero_row, 0)

  def zero_chunk(i, carry):
    b = nbase + i * CH
    pltpu.sync_copy(exc_v, acc_sp.at[pl.ds(b, CH)])
    return carry
  lax.fori_loop(0, nchunks, zero_chunk, 0)

  plsc.subcore_barrier()

  ebase = s * EPT

  def edge_chunk(i, carry):
    b = ebase + i * CH
    pltpu.sync_copy(ep.at[pl.ds(b, CH)], idx_src)
    pltpu.sync_copy(ep.at[pl.ds(EP + b, CH)], idx_dst.at[0])
    pltpu.sync_copy(ep.at[pl.ds(2 * EP + b, CH)], idx_f)
    pltpu.async_copy(hin.at[idx_src], hsrc_v, sem).wait()
    pltpu.async_copy(fused.at[idx_f], emb_v, sem).wait()

    def compute_row(r, carry2):
      for k in range(HH // 16):
        sl = pl.ds(16 * k, 16)
        hsl = pl.ds(cb + 16 * k, 16)
        m = jnp.maximum(hsrc_v[r, hsl] + emb_v[r, hsl], 0.0) + EPS
        e = jnp.exp(m)
        exc_v[r, sl] = e
        exc_v[r, pl.ds(HH + 16 * k, 16)] = e * m
      return carry2
    lax.fori_loop(0, CH, compute_row, 0)

    pltpu.async_copy(exc_v, acc_sp.at[idx_dst.at[0]], sem, add=True).wait()
    return carry
  lax.fori_loop(0, NECHUNK, edge_chunk, 0)

  plsc.subcore_barrier()

  def div_chunk(i, carry):
    b = nbase + i * CH
    pltpu.sync_copy(acc_sp.at[pl.ds(b, CH)], hsrc_v)

    def div_row(r, carry2):
      for k in range(HH // 16):
        sl = pl.ds(16 * k, 16)
        out_v[r, sl] = (hsrc_v[r, pl.ds(HH + 16 * k, 16)]
                        / (hsrc_v[r, sl] + 1e-16))
      return carry2
    lax.fori_loop(0, CH, div_row, 0)
    pltpu.sync_copy(out_v, agg_out.at[pl.ds(c * NN + b, CH)])
    return carry
  lax.fori_loop(0, nchunks, div_chunk, 0)


def _sc_aggregate(name, hin, fused, ep):
  f = pl.kernel(
      _sc_body,
      out_type=jax.ShapeDtypeStruct((2 * NN, HH), jnp.float32),
      mesh=plsc.VectorSubcoreMesh(core_axis_name="c", subcore_axis_name="s",
                                  num_cores=2, num_subcores=NSUB),
      scratch_types=[
          pltpu.VMEM_SHARED((NR, H), jnp.float32),
          pltpu.VMEM((CH,), jnp.int32),
          pltpu.VMEM((1, CH), jnp.int32),
          pltpu.VMEM((CH,), jnp.int32),
          pltpu.VMEM((CH, H), jnp.float32),
          pltpu.VMEM((CH, H), jnp.float32),
          pltpu.VMEM((CH, H), jnp.float32),
          pltpu.VMEM((CH, HH), jnp.float32),
          pltpu.SemaphoreType.DMA,
      ],
      name=name,
  )
  return f(hin, fused, ep)


def _ln(h, g, b):
  mu = jnp.mean(h, axis=-1, keepdims=True)
  var = jnp.mean(jnp.square(h - mu), axis=-1, keepdims=True)
  return (h - mu) / jnp.sqrt(var + 1e-5) * g + b


def _tc_update_body(has_res, h2_ref, agg2_ref, W_ref, b_ref, g_ref,
                    beta_ref, *rest):
  if has_res:
    res_ref, h_out_ref, x2_out_ref = rest
  else:
    h_out_ref, x2_out_ref = rest
  agg = jnp.concatenate([agg2_ref[pl.ds(0, NN), :], agg2_ref[pl.ds(NN, NN), :]],
                        axis=-1)
  t = (h2_ref[...] + agg) @ W_ref[...] + b_ref[...]
  if has_res:
    t = t + res_ref[...]
  h_out_ref[...] = t
  x2_out_ref[...] = jax.nn.relu(_ln(t, g_ref[...], beta_ref[...]))


def _tc_update(h2, agg2, W, b, g, beta, res=None):
  args = [h2, agg2, W, b, g, beta]
  if res is not None:
    args.append(res)
  return pl.pallas_call(
      functools.partial(_tc_update_body, res is not None),
      out_shape=[
          jax.ShapeDtypeStruct((NN, H), jnp.float32),
          jax.ShapeDtypeStruct((NN, H), jnp.float32),
      ],
  )(*args)


def _tc_final_body(h2_ref, agg2_ref, W_ref, b_ref, res_ref, g_ref,
                   beta_ref, batch_ref, out_ref):
  agg = jnp.concatenate([agg2_ref[pl.ds(0, NN), :], agg2_ref[pl.ds(NN, NN), :]],
                        axis=-1)
  t = (h2_ref[...] + agg) @ W_ref[...] + b_ref[...] + res_ref[...]
  hn = _ln(t, g_ref[...], beta_ref[...])[:N]
  onehot = (batch_ref[...] == lax.broadcasted_iota(jnp.int32, (N, G), 1)
            ).astype(jnp.float32)
  sums = lax.dot_general(onehot, hn, (((0,), (0,)), ((), ())),
                         preferred_element_type=jnp.float32)
  counts = lax.dot_general(onehot, jnp.ones((N, 1), jnp.float32),
                           (((0,), (0,)), ((), ())),
                           preferred_element_type=jnp.float32)
  out_ref[...] = sums / jnp.maximum(counts, 1.0)


def _tc_final(h2, agg2, W, b, res, g, beta, batch2d):
  return pl.pallas_call(
      _tc_final_body,
      out_shape=jax.ShapeDtypeStruct((G, H), jnp.float32),
  )(h2, agg2, W, b, res, g, beta, batch2d)


def kernel(x, edge_index, edge_attr, batch, atom_tables, bond_tables,
           gcn_W, gcn_b, norm_gamma, norm_beta):
  i32 = jnp.int32
  xT = jnp.transpose(x).astype(i32)
  eaT = jnp.transpose(edge_attr).astype(i32)
  ei = edge_index.astype(i32)
  batch2d = batch.astype(i32).reshape(N, 1)
  W = [gcn_W[i] for i in range(3)]
  b = [gcn_b[i].reshape(1, H) for i in range(3)]
  g = [norm_gamma[i].reshape(1, H) for i in range(3)]
  beta = [norm_beta[i].reshape(1, H) for i in range(3)]

  h0, fused, ep = _tc_prep(xT, ei, eaT, atom_tables, bond_tables)

  agg0 = _sc_aggregate("sc_agg_0", h0, fused, ep)
  h_a, xc2 = _tc_update(h0, agg0, W[0], b[0], g[0], beta[0])
  agg1 = _sc_aggregate("sc_agg_1", xc2, fused, ep)
  h_b, xc3 = _tc_update(xc2, agg1, W[1], b[1], g[1], beta[1], res=h_a)
  agg2 = _sc_aggregate("sc_agg_2", xc3, fused, ep)
  return _tc_final(xc3, agg2, W[2], b[2], h_b, g[2], beta[2], batch2d)


# speedup vs baseline: 2.4732x; 1.3815x over previous
import functools

import jax
import jax.numpy as jnp
from jax import lax
from jax.experimental import pallas as pl
from jax.experimental.pallas import tpu as pltpu
from jax.experimental.pallas import tpu_sc as plsc

N = 10000
E = 320000
H = 128
HH = 64
G = 64
EPS = 1e-7

NSUB = 16
CH = 64
NN = 10240
EP = 327680
NPT = NN // NSUB
NCHUNK_N = NPT // CH
EPT = EP // NSUB
NECHUNK = EPT // CH
NR = 10048


def _tc_prep_body(xT_ref, ei_ref, eaT_ref, at_ref, bd_ref,
                  h0_ref, fused_ref, ep_ref):
  h = jnp.zeros((N, H), jnp.float32)
  for f in range(3):
    xf = xT_ref[f].reshape(N, 1)
    onehot = (xf == lax.broadcasted_iota(jnp.int32, (N, 64), 1)
              ).astype(jnp.float32)
    h = h + lax.dot_general(onehot, at_ref[f], (((1,), (0,)), ((), ())),
                            preferred_element_type=jnp.float32)
  h0_ref[pl.ds(0, N), :] = h
  h0_ref[pl.ds(N, NN - N), :] = jnp.zeros((NN - N, H), jnp.float32)

  r = lax.broadcasted_iota(jnp.int32, (512, 1), 0)
  fu = jnp.zeros((512, H), jnp.float32)
  for f, sel in enumerate((r // 64, (r // 8) % 8, r % 8)):
    onehot = (sel == lax.broadcasted_iota(jnp.int32, (512, 8), 1)
              ).astype(jnp.float32)
    fu = fu + lax.dot_general(onehot, bd_ref[f], (((1,), (0,)), ((), ())),
                              preferred_element_type=jnp.float32)
  fused_ref[...] = fu

  ep_ref[pl.ds(0, E)] = ei_ref[0]
  ep_ref[pl.ds(E, EP - E)] = jnp.zeros((EP - E,), jnp.int32)
  ep_ref[pl.ds(EP, E)] = ei_ref[1]
  ep_ref[pl.ds(EP + E, EP - E)] = jnp.full((EP - E,), N, jnp.int32)
  ep_ref[pl.ds(2 * EP, E)] = eaT_ref[0] * 64 + eaT_ref[1] * 8 + eaT_ref[2]
  ep_ref[pl.ds(2 * EP + E, EP - E)] = jnp.zeros((EP - E,), jnp.int32)


def _tc_prep(xT, ei, eaT, at, bd):
  return pl.pallas_call(
      _tc_prep_body,
      out_shape=[
          jax.ShapeDtypeStruct((NN, H), jnp.float32),
          jax.ShapeDtypeStruct((512, H), jnp.float32),
          jax.ShapeDtypeStruct((3 * EP,), jnp.int32),
      ],
  )(xT, ei, eaT, at, bd)


def _sc_body(hin, fused, ep,
             agg_out,
             acc_sp,
             idx_srcA, idx_dstA, idx_fA,
             idx_srcB, idx_dstB, idx_fB,
             hsrc_v, emb_v, excA, excB, seml, semg, sems):
  c = lax.axis_index("c")
  s = lax.axis_index("s")
  cb = c * HH
  nbase = s * NPT
  nchunks = jnp.where(s == NSUB - 1, NCHUNK_N - 3, NCHUNK_N)

  def zero_row(i, carry):
    for k in range(H // 16):
      sl = pl.ds(16 * k, 16)
      excA[i, sl] = jnp.zeros((16,), jnp.float32)
    return carry
  lax.fori_loop(0, CH, zero_row, 0)

  def zero_chunk(i, carry):
    b = nbase + i * CH
    pltpu.sync_copy(excA, acc_sp.at[pl.ds(b, CH)])
    return carry
  lax.fori_loop(0, nchunks, zero_chunk, 0)

  plsc.subcore_barrier()

  ebase = s * EPT

  def compute_chunk(exc):
    def compute_row(r, carry2):
      for k in range(HH // 16):
        sl = pl.ds(16 * k, 16)
        hsl = pl.ds(cb + 16 * k, 16)
        m = jnp.maximum(hsrc_v[r, hsl] + emb_v[r, hsl], 0.0) + EPS
        e = jnp.exp(m)
        exc[r, sl] = e
        exc[r, pl.ds(HH + 16 * k, 16)] = e * m
      return carry2
    lax.fori_loop(0, CH, compute_row, 0)

  def edge_pair(g, carry):
    bA = ebase + (2 * g) * CH
    bB = bA + CH
    lA0 = pltpu.async_copy(ep.at[pl.ds(bA, CH)], idx_srcA, seml)
    lA1 = pltpu.async_copy(ep.at[pl.ds(EP + bA, CH)], idx_dstA.at[0], seml)
    lA2 = pltpu.async_copy(ep.at[pl.ds(2 * EP + bA, CH)], idx_fA, seml)
    lA0.wait()
    lA1.wait()
    lA2.wait()
    gA0 = pltpu.async_copy(hin.at[idx_srcA], hsrc_v, semg)
    gA1 = pltpu.async_copy(fused.at[idx_fA], emb_v, semg)
    lB0 = pltpu.async_copy(ep.at[pl.ds(bB, CH)], idx_srcB, seml)
    lB1 = pltpu.async_copy(ep.at[pl.ds(EP + bB, CH)], idx_dstB.at[0], seml)
    lB2 = pltpu.async_copy(ep.at[pl.ds(2 * EP + bB, CH)], idx_fB, seml)
    gA0.wait()
    gA1.wait()
    compute_chunk(excA)
    sA = pltpu.async_copy(excA, acc_sp.at[idx_dstA.at[0]], sems, add=True)
    lB0.wait()
    lB1.wait()
    lB2.wait()
    gB0 = pltpu.async_copy(hin.at[idx_srcB], hsrc_v, semg)
    gB1 = pltpu.async_copy(fused.at[idx_fB], emb_v, semg)
    gB0.wait()
    gB1.wait()
    compute_chunk(excB)
    sA.wait()
    sB = pltpu.async_copy(excB, acc_sp.at[idx_dstB.at[0]], sems, add=True)
    sB.wait()
    return carry
  lax.fori_loop(0, NECHUNK // 2, edge_pair, 0)

  plsc.subcore_barrier()

  def div_chunk(i, carry):
    b = nbase + i * CH
    pltpu.sync_copy(acc_sp.at[pl.ds(b, CH)], hsrc_v)

    def div_row(r, carry2):
      for k in range(HH // 16):
        sl = pl.ds(16 * k, 16)
        excA[r, sl] = (hsrc_v[r, pl.ds(HH + 16 * k, 16)]
                       / (hsrc_v[r, sl] + 1e-16))
      return carry2
    lax.fori_loop(0, CH, div_row, 0)
    pltpu.sync_copy(excA, agg_out.at[pl.ds(c * NN + b, CH)])
    return carry
  lax.fori_loop(0, nchunks, div_chunk, 0)


def _sc_aggregate(name, hin, fused, ep):
  f = pl.kernel(
      _sc_body,
      out_type=jax.ShapeDtypeStruct((2 * NN, H), jnp.float32),
      mesh=plsc.VectorSubcoreMesh(core_axis_name="c", subcore_axis_name="s",
                                  num_cores=2, num_subcores=NSUB),
      scratch_types=[
          pltpu.VMEM_SHARED((NR, H), jnp.float32),
          pltpu.VMEM((CH,), jnp.int32),
          pltpu.VMEM((1, CH), jnp.int32),
          pltpu.VMEM((CH,), jnp.int32),
          pltpu.VMEM((CH,), jnp.int32),
          pltpu.VMEM((1, CH), jnp.int32),
          pltpu.VMEM((CH,), jnp.int32),
          pltpu.VMEM((CH, H), jnp.float32),
          pltpu.VMEM((CH, H), jnp.float32),
          pltpu.VMEM((CH, H), jnp.float32),
          pltpu.VMEM((CH, H), jnp.float32),
          pltpu.SemaphoreType.DMA,
          pltpu.SemaphoreType.DMA,
          pltpu.SemaphoreType.DMA,
      ],
      name=name,
  )
  return f(hin, fused, ep)


def _ln(h, g, b):
  mu = jnp.mean(h, axis=-1, keepdims=True)
  var = jnp.mean(jnp.square(h - mu), axis=-1, keepdims=True)
  return (h - mu) / jnp.sqrt(var + 1e-5) * g + b


def _tc_update_body(has_res, h2_ref, agg2_ref, W_ref, b_ref, g_ref,
                    beta_ref, *rest):
  if has_res:
    res_ref, h_out_ref, x2_out_ref = rest
  else:
    h_out_ref, x2_out_ref = rest
  a2 = agg2_ref[...]
  agg = jnp.concatenate([a2[:NN, :HH], a2[NN:, :HH]], axis=-1)
  t = (h2_ref[...] + agg) @ W_ref[...] + b_ref[...]
  if has_res:
    t = t + res_ref[...]
  h_out_ref[...] = t
  x2_out_ref[...] = jax.nn.relu(_ln(t, g_ref[...], beta_ref[...]))


def _tc_update(h2, agg2, W, b, g, beta, res=None):
  args = [h2, agg2, W, b, g, beta]
  if res is not None:
    args.append(res)
  return pl.pallas_call(
      functools.partial(_tc_update_body, res is not None),
      out_shape=[
          jax.ShapeDtypeStruct((NN, H), jnp.float32),
          jax.ShapeDtypeStruct((NN, H), jnp.float32),
      ],
  )(*args)


def _tc_final_body(h2_ref, agg2_ref, W_ref, b_ref, res_ref, g_ref,
                   beta_ref, batch_ref, out_ref):
  a2 = agg2_ref[...]
  agg = jnp.concatenate([a2[:NN, :HH], a2[NN:, :HH]], axis=-1)
  t = (h2_ref[...] + agg) @ W_ref[...] + b_ref[...] + res_ref[...]
  hn = _ln(t, g_ref[...], beta_ref[...])[:N]
  onehot = (batch_ref[...] == lax.broadcasted_iota(jnp.int32, (N, G), 1)
            ).astype(jnp.float32)
  sums = lax.dot_general(onehot, hn, (((0,), (0,)), ((), ())),
                         preferred_element_type=jnp.float32)
  counts = lax.dot_general(onehot, jnp.ones((N, 1), jnp.float32),
                           (((0,), (0,)), ((), ())),
                           preferred_element_type=jnp.float32)
  out_ref[...] = sums / jnp.maximum(counts, 1.0)


def _tc_final(h2, agg2, W, b, res, g, beta, batch2d):
  return pl.pallas_call(
      _tc_final_body,
      out_shape=jax.ShapeDtypeStruct((G, H), jnp.float32),
  )(h2, agg2, W, b, res, g, beta, batch2d)


def kernel(x, edge_index, edge_attr, batch, atom_tables, bond_tables,
           gcn_W, gcn_b, norm_gamma, norm_beta):
  i32 = jnp.int32
  xT = jnp.transpose(x).astype(i32)
  eaT = jnp.transpose(edge_attr).astype(i32)
  ei = edge_index.astype(i32)
  batch2d = batch.astype(i32).reshape(N, 1)
  W = [gcn_W[i] for i in range(3)]
  b = [gcn_b[i].reshape(1, H) for i in range(3)]
  g = [norm_gamma[i].reshape(1, H) for i in range(3)]
  beta = [norm_beta[i].reshape(1, H) for i in range(3)]

  h0, fused, ep = _tc_prep(xT, ei, eaT, atom_tables, bond_tables)

  agg0 = _sc_aggregate("sc_agg_0", h0, fused, ep)
  h_a, xc2 = _tc_update(h0, agg0, W[0], b[0], g[0], beta[0])
  agg1 = _sc_aggregate("sc_agg_1", xc2, fused, ep)
  h_b, xc3 = _tc_update(xc2, agg1, W[1], b[1], g[1], beta[1], res=h_a)
  agg2 = _sc_aggregate("sc_agg_2", xc3, fused, ep)
  return _tc_final(xc3, agg2, W[2], b[2], h_b, g[2], beta[2], batch2d)


# speedup vs baseline: 2.8151x; 1.1383x over previous
import functools

import jax
import jax.numpy as jnp
from jax import lax
from jax.experimental import pallas as pl
from jax.experimental.pallas import tpu as pltpu
from jax.experimental.pallas import tpu_sc as plsc

N = 10000
E = 320000
H = 128
HH = 64
G = 64
EPS = 1e-7

NSUB = 16
CH = 40
NN = 10240
EP = 327680
NPT = NN // NSUB
NCHUNK_N = NPT // CH
EPT = EP // NSUB
NECHUNK = EPT // CH
NR = 10080


def _tc_prep_body(xT_ref, ei_ref, eaT_ref, at_ref, bd_ref,
                  h0_ref, fused_ref, ep_ref):
  h = jnp.zeros((N, H), jnp.float32)
  for f in range(3):
    xf = xT_ref[f].reshape(N, 1)
    onehot = (xf == lax.broadcasted_iota(jnp.int32, (N, 64), 1)
              ).astype(jnp.float32)
    h = h + lax.dot_general(onehot, at_ref[f], (((1,), (0,)), ((), ())),
                            preferred_element_type=jnp.float32)
  h0_ref[pl.ds(0, N), :] = h
  h0_ref[pl.ds(N, NN - N), :] = jnp.zeros((NN - N, H), jnp.float32)

  r = lax.broadcasted_iota(jnp.int32, (512, 1), 0)
  fu = jnp.zeros((512, H), jnp.float32)
  for f, sel in enumerate((r // 64, (r // 8) % 8, r % 8)):
    onehot = (sel == lax.broadcasted_iota(jnp.int32, (512, 8), 1)
              ).astype(jnp.float32)
    fu = fu + lax.dot_general(onehot, bd_ref[f], (((1,), (0,)), ((), ())),
                              preferred_element_type=jnp.float32)
  fused_ref[...] = fu

  ep_ref[pl.ds(0, E)] = ei_ref[0]
  ep_ref[pl.ds(E, EP - E)] = jnp.zeros((EP - E,), jnp.int32)
  ep_ref[pl.ds(EP, E)] = ei_ref[1]
  ep_ref[pl.ds(EP + E, EP - E)] = jnp.full((EP - E,), N, jnp.int32)
  ep_ref[pl.ds(2 * EP, E)] = eaT_ref[0] * 64 + eaT_ref[1] * 8 + eaT_ref[2]
  ep_ref[pl.ds(2 * EP + E, EP - E)] = jnp.zeros((EP - E,), jnp.int32)


def _tc_prep(xT, ei, eaT, at, bd):
  return pl.pallas_call(
      _tc_prep_body,
      out_shape=[
          jax.ShapeDtypeStruct((NN, H), jnp.float32),
          jax.ShapeDtypeStruct((512, H), jnp.float32),
          jax.ShapeDtypeStruct((3 * EP,), jnp.int32),
      ],
  )(xT, ei, eaT, at, bd)


def _sc_body(hin, fused, ep,
             agg_out,
             acc_sp,
             idx_srcA, idx_dstA, idx_fA,
             idx_srcB, idx_dstB, idx_fB,
             hsrcA, embA, hsrcB, embB, excA, excB, seml, semg, sems):
  c = lax.axis_index("c")
  s = lax.axis_index("s")
  cb = c * HH
  nbase = s * NPT
  nchunks = jnp.where(s == NSUB - 1, NCHUNK_N - 4, NCHUNK_N)

  def zero_row(i, carry):
    for k in range(H // 16):
      sl = pl.ds(16 * k, 16)
      excA[i, sl] = jnp.zeros((16,), jnp.float32)
    return carry
  lax.fori_loop(0, CH, zero_row, 0)

  def zero_chunk(i, carry):
    b = nbase + i * CH
    pltpu.sync_copy(excA, acc_sp.at[pl.ds(b, CH)])
    return carry
  lax.fori_loop(0, nchunks, zero_chunk, 0)

  plsc.subcore_barrier()

  ebase = s * EPT

  def compute_chunk(hsrc, emb, exc):
    def compute_row(r, carry2):
      for k in range(HH // 16):
        sl = pl.ds(16 * k, 16)
        hsl = pl.ds(cb + 16 * k, 16)
        m = jnp.maximum(hsrc[r, hsl] + emb[r, hsl], 0.0) + EPS
        e = jnp.exp(m)
        exc[r, sl] = e
        exc[r, pl.ds(HH + 16 * k, 16)] = e * m
      return carry2
    lax.fori_loop(0, CH, compute_row, 0)

  def edge_pair(g, carry):
    bA = ebase + (2 * g) * CH
    bB = bA + CH
    lA0 = pltpu.async_copy(ep.at[pl.ds(bA, CH)], idx_srcA, seml)
    lA1 = pltpu.async_copy(ep.at[pl.ds(EP + bA, CH)], idx_dstA.at[0], seml)
    lA2 = pltpu.async_copy(ep.at[pl.ds(2 * EP + bA, CH)], idx_fA, seml)
    lB0 = pltpu.async_copy(ep.at[pl.ds(bB, CH)], idx_srcB, seml)
    lB1 = pltpu.async_copy(ep.at[pl.ds(EP + bB, CH)], idx_dstB.at[0], seml)
    lB2 = pltpu.async_copy(ep.at[pl.ds(2 * EP + bB, CH)], idx_fB, seml)
    lA0.wait()
    lA1.wait()
    lA2.wait()
    gA0 = pltpu.async_copy(hin.at[idx_srcA], hsrcA, semg)
    gA1 = pltpu.async_copy(fused.at[idx_fA], embA, semg)
    lB0.wait()
    lB1.wait()
    lB2.wait()
    gB0 = pltpu.async_copy(hin.at[idx_srcB], hsrcB, semg)
    gB1 = pltpu.async_copy(fused.at[idx_fB], embB, semg)
    gA0.wait()
    gA1.wait()
    compute_chunk(hsrcA, embA, excA)
    sA = pltpu.async_copy(excA, acc_sp.at[idx_dstA.at[0]], sems, add=True)
    gB0.wait()
    gB1.wait()
    compute_chunk(hsrcB, embB, excB)
    sA.wait()
    sB = pltpu.async_copy(excB, acc_sp.at[idx_dstB.at[0]], sems, add=True)
    sB.wait()
    return carry
  lax.fori_loop(0, NECHUNK // 2, edge_pair, 0)

  plsc.subcore_barrier()

  def div_chunk(i, carry):
    b = nbase + i * CH
    pltpu.sync_copy(acc_sp.at[pl.ds(b, CH)], hsrcA)

    def div_row(r, carry2):
      for k in range(HH // 16):
        sl = pl.ds(16 * k, 16)
        excA[r, sl] = (hsrcA[r, pl.ds(HH + 16 * k, 16)]
                       / (hsrcA[r, sl] + 1e-16))
      return carry2
    lax.fori_loop(0, CH, div_row, 0)
    pltpu.sync_copy(excA, agg_out.at[pl.ds(c * NN + b, CH)])
    return carry
  lax.fori_loop(0, nchunks, div_chunk, 0)


def _sc_aggregate(name, hin, fused, ep):
  f = pl.kernel(
      _sc_body,
      out_type=jax.ShapeDtypeStruct((2 * NN, H), jnp.float32),
      mesh=plsc.VectorSubcoreMesh(core_axis_name="c", subcore_axis_name="s",
                                  num_cores=2, num_subcores=NSUB),
      scratch_types=[
          pltpu.VMEM_SHARED((NR, H), jnp.float32),
          pltpu.VMEM((CH,), jnp.int32),
          pltpu.VMEM((1, CH), jnp.int32),
          pltpu.VMEM((CH,), jnp.int32),
          pltpu.VMEM((CH,), jnp.int32),
          pltpu.VMEM((1, CH), jnp.int32),
          pltpu.VMEM((CH,), jnp.int32),
          pltpu.VMEM((CH, H), jnp.float32),
          pltpu.VMEM((CH, H), jnp.float32),
          pltpu.VMEM((CH, H), jnp.float32),
          pltpu.VMEM((CH, H), jnp.float32),
          pltpu.VMEM((CH, H), jnp.float32),
          pltpu.VMEM((CH, H), jnp.float32),
          pltpu.SemaphoreType.DMA,
          pltpu.SemaphoreType.DMA,
          pltpu.SemaphoreType.DMA,
      ],
      name=name,
  )
  return f(hin, fused, ep)


def _ln(h, g, b):
  mu = jnp.mean(h, axis=-1, keepdims=True)
  var = jnp.mean(jnp.square(h - mu), axis=-1, keepdims=True)
  return (h - mu) / jnp.sqrt(var + 1e-5) * g + b


def _tc_update_body(has_res, h2_ref, agg2_ref, W_ref, b_ref, g_ref,
                    beta_ref, *rest):
  if has_res:
    res_ref, h_out_ref, x2_out_ref = rest
  else:
    h_out_ref, x2_out_ref = rest
  a2 = agg2_ref[...]
  agg = jnp.concatenate([a2[:NN, :HH], a2[NN:, :HH]], axis=-1)
  t = (h2_ref[...] + agg) @ W_ref[...] + b_ref[...]
  if has_res:
    t = t + res_ref[...]
  h_out_ref[...] = t
  x2_out_ref[...] = jax.nn.relu(_ln(t, g_ref[...], beta_ref[...]))


def _tc_update(h2, agg2, W, b, g, beta, res=None):
  args = [h2, agg2, W, b, g, beta]
  if res is not None:
    args.append(res)
  return pl.pallas_call(
      functools.partial(_tc_update_body, res is not None),
      out_shape=[
          jax.ShapeDtypeStruct((NN, H), jnp.float32),
          jax.ShapeDtypeStruct((NN, H), jnp.float32),
      ],
  )(*args)


def _tc_final_body(h2_ref, agg2_ref, W_ref, b_ref, res_ref, g_ref,
                   beta_ref, batch_ref, out_ref):
  a2 = agg2_ref[...]
  agg = jnp.concatenate([a2[:NN, :HH], a2[NN:, :HH]], axis=-1)
  t = (h2_ref[...] + agg) @ W_ref[...] + b_ref[...] + res_ref[...]
  hn = _ln(t, g_ref[...], beta_ref[...])[:N]
  onehot = (batch_ref[...] == lax.broadcasted_iota(jnp.int32, (N, G), 1)
            ).astype(jnp.float32)
  sums = lax.dot_general(onehot, hn, (((0,), (0,)), ((), ())),
                         preferred_element_type=jnp.float32)
  counts = lax.dot_general(onehot, jnp.ones((N, 1), jnp.float32),
                           (((0,), (0,)), ((), ())),
                           preferred_element_type=jnp.float32)
  out_ref[...] = sums / jnp.maximum(counts, 1.0)


def _tc_final(h2, agg2, W, b, res, g, beta, batch2d):
  return pl.pallas_call(
      _tc_final_body,
      out_shape=jax.ShapeDtypeStruct((G, H), jnp.float32),
  )(h2, agg2, W, b, res, g, beta, batch2d)


def kernel(x, edge_index, edge_attr, batch, atom_tables, bond_tables,
           gcn_W, gcn_b, norm_gamma, norm_beta):
  i32 = jnp.int32
  xT = jnp.transpose(x).astype(i32)
  eaT = jnp.transpose(edge_attr).astype(i32)
  ei = edge_index.astype(i32)
  batch2d = batch.astype(i32).reshape(N, 1)
  W = [gcn_W[i] for i in range(3)]
  b = [gcn_b[i].reshape(1, H) for i in range(3)]
  g = [norm_gamma[i].reshape(1, H) for i in range(3)]
  beta = [norm_beta[i].reshape(1, H) for i in range(3)]

  h0, fused, ep = _tc_prep(xT, ei, eaT, atom_tables, bond_tables)

  agg0 = _sc_aggregate("sc_agg_0", h0, fused, ep)
  h_a, xc2 = _tc_update(h0, agg0, W[0], b[0], g[0], beta[0])
  agg1 = _sc_aggregate("sc_agg_1", xc2, fused, ep)
  h_b, xc3 = _tc_update(xc2, agg1, W[1], b[1], g[1], beta[1], res=h_a)
  agg2 = _sc_aggregate("sc_agg_2", xc3, fused, ep)
  return _tc_final(xc3, agg2, W[2], b[2], h_b, g[2], beta[2], batch2d)


# speedup vs baseline: 2.9104x; 1.0338x over previous
import functools

import jax
import jax.numpy as jnp
from jax import lax
from jax.experimental import pallas as pl
from jax.experimental.pallas import tpu as pltpu
from jax.experimental.pallas import tpu_sc as plsc

N = 10000
E = 320000
H = 128
HH = 64
G = 64
EPS = 1e-7

NSUB = 16
CH = 40
NN = 10240
EP = 327680
NPT = NN // NSUB
NCHUNK_N = NPT // CH
EPT = EP // NSUB
NECHUNK = EPT // CH
NR = 10080


def _tc_prep_body(xT_ref, ei_ref, eaT_ref, at_ref, bd_ref,
                  h0_ref, fused_ref, ep_ref):
  h = jnp.zeros((N, H), jnp.float32)
  for f in range(3):
    xf = xT_ref[f].reshape(N, 1)
    onehot = (xf == lax.broadcasted_iota(jnp.int32, (N, 64), 1)
              ).astype(jnp.float32)
    h = h + lax.dot_general(onehot, at_ref[f], (((1,), (0,)), ((), ())),
                            preferred_element_type=jnp.float32)
  h0_ref[pl.ds(0, N), :] = h
  h0_ref[pl.ds(N, NN - N), :] = jnp.zeros((NN - N, H), jnp.float32)

  r = lax.broadcasted_iota(jnp.int32, (512, 1), 0)
  fu = jnp.zeros((512, H), jnp.float32)
  for f, sel in enumerate((r // 64, (r // 8) % 8, r % 8)):
    onehot = (sel == lax.broadcasted_iota(jnp.int32, (512, 8), 1)
              ).astype(jnp.float32)
    fu = fu + lax.dot_general(onehot, bd_ref[f], (((1,), (0,)), ((), ())),
                              preferred_element_type=jnp.float32)
  fused_ref[...] = fu

  ep_ref[pl.ds(0, E)] = ei_ref[0]
  ep_ref[pl.ds(E, EP - E)] = jnp.zeros((EP - E,), jnp.int32)
  ep_ref[pl.ds(EP, E)] = ei_ref[1]
  ep_ref[pl.ds(EP + E, EP - E)] = jnp.full((EP - E,), N, jnp.int32)
  ep_ref[pl.ds(2 * EP, E)] = eaT_ref[0] * 64 + eaT_ref[1] * 8 + eaT_ref[2]
  ep_ref[pl.ds(2 * EP + E, EP - E)] = jnp.zeros((EP - E,), jnp.int32)


def _tc_prep(xT, ei, eaT, at, bd):
  return pl.pallas_call(
      _tc_prep_body,
      out_shape=[
          jax.ShapeDtypeStruct((NN, H), jnp.float32),
          jax.ShapeDtypeStruct((512, H), jnp.float32),
          jax.ShapeDtypeStruct((3 * EP,), jnp.int32),
      ],
  )(xT, ei, eaT, at, bd)


def _sc_body(hin, fused, ep,
             agg_out,
             acc_sp,
             idx_srcA, idx_dstA, idx_fA,
             idx_srcB, idx_dstB, idx_fB,
             hsrcA, embA, hsrcB, embB, excA, excB, seml, semg, sems):
  c = lax.axis_index("c")
  s = lax.axis_index("s")
  cb = c * HH
  nbase = s * NPT
  nchunks = jnp.where(s == NSUB - 1, NCHUNK_N - 4, NCHUNK_N)

  def zero_row(i, carry):
    for k in range(H // 16):
      sl = pl.ds(16 * k, 16)
      excA[i, sl] = jnp.zeros((16,), jnp.float32)
    return carry
  lax.fori_loop(0, CH, zero_row, 0)

  def zero_chunk(i, carry):
    b = nbase + i * CH
    pltpu.sync_copy(excA, acc_sp.at[pl.ds(b, CH)])
    return carry
  lax.fori_loop(0, nchunks, zero_chunk, 0)

  plsc.subcore_barrier()

  ebase = s * EPT

  def compute_chunk(hsrc, emb, exc):
    def compute_row(r, carry2):
      for k in range(HH // 16):
        sl = pl.ds(16 * k, 16)
        hsl = pl.ds(cb + 16 * k, 16)
        m = jnp.maximum(hsrc[r, hsl] + emb[r, hsl], 0.0) + EPS
        e = jnp.exp(m)
        exc[r, sl] = e
        exc[r, pl.ds(HH + 16 * k, 16)] = e * m
      return carry2
    lax.fori_loop(0, CH, compute_row, 0)

  def edge_pair(g, carry):
    bA = ebase + (2 * g) * CH
    bB = bA + CH
    lA0 = pltpu.async_copy(ep.at[pl.ds(bA, CH)], idx_srcA, seml)
    lA1 = pltpu.async_copy(ep.at[pl.ds(EP + bA, CH)], idx_dstA.at[0], seml)
    lA2 = pltpu.async_copy(ep.at[pl.ds(2 * EP + bA, CH)], idx_fA, seml)
    lB0 = pltpu.async_copy(ep.at[pl.ds(bB, CH)], idx_srcB, seml)
    lB1 = pltpu.async_copy(ep.at[pl.ds(EP + bB, CH)], idx_dstB.at[0], seml)
    lB2 = pltpu.async_copy(ep.at[pl.ds(2 * EP + bB, CH)], idx_fB, seml)
    lA0.wait()
    lA1.wait()
    lA2.wait()
    gA0 = pltpu.async_copy(hin.at[idx_srcA], hsrcA, semg)
    gA1 = pltpu.async_copy(fused.at[idx_fA], embA, semg)
    lB0.wait()
    lB1.wait()
    lB2.wait()
    gB0 = pltpu.async_copy(hin.at[idx_srcB], hsrcB, semg)
    gB1 = pltpu.async_copy(fused.at[idx_fB], embB, semg)
    gA0.wait()
    gA1.wait()

    @pl.when(g != 0)
    def _drain_prev():
      pltpu.make_async_copy(excA, acc_sp.at[idx_dstA.at[0]], sems).wait()
      pltpu.make_async_copy(excB, acc_sp.at[idx_dstB.at[0]], sems).wait()

    compute_chunk(hsrcA, embA, excA)
    pltpu.async_copy(excA, acc_sp.at[idx_dstA.at[0]], sems, add=True)
    gB0.wait()
    gB1.wait()
    compute_chunk(hsrcB, embB, excB)
    pltpu.async_copy(excB, acc_sp.at[idx_dstB.at[0]], sems, add=True)
    return carry
  lax.fori_loop(0, NECHUNK // 2, edge_pair, 0)
  pltpu.make_async_copy(excA, acc_sp.at[idx_dstA.at[0]], sems).wait()
  pltpu.make_async_copy(excB, acc_sp.at[idx_dstB.at[0]], sems).wait()

  plsc.subcore_barrier()

  def div_chunk(i, carry):
    b = nbase + i * CH
    pltpu.sync_copy(acc_sp.at[pl.ds(b, CH)], hsrcA)

    def div_row(r, carry2):
      for k in range(HH // 16):
        sl = pl.ds(16 * k, 16)
        excA[r, sl] = (hsrcA[r, pl.ds(HH + 16 * k, 16)]
                       / (hsrcA[r, sl] + 1e-16))
      return carry2
    lax.fori_loop(0, CH, div_row, 0)
    pltpu.sync_copy(excA, agg_out.at[pl.ds(c * NN + b, CH)])
    return carry
  lax.fori_loop(0, nchunks, div_chunk, 0)


def _sc_aggregate(name, hin, fused, ep):
  f = pl.kernel(
      _sc_body,
      out_type=jax.ShapeDtypeStruct((2 * NN, H), jnp.float32),
      mesh=plsc.VectorSubcoreMesh(core_axis_name="c", subcore_axis_name="s",
                                  num_cores=2, num_subcores=NSUB),
      scratch_types=[
          pltpu.VMEM_SHARED((NR, H), jnp.float32),
          pltpu.VMEM((CH,), jnp.int32),
          pltpu.VMEM((1, CH), jnp.int32),
          pltpu.VMEM((CH,), jnp.int32),
          pltpu.VMEM((CH,), jnp.int32),
          pltpu.VMEM((1, CH), jnp.int32),
          pltpu.VMEM((CH,), jnp.int32),
          pltpu.VMEM((CH, H), jnp.float32),
          pltpu.VMEM((CH, H), jnp.float32),
          pltpu.VMEM((CH, H), jnp.float32),
          pltpu.VMEM((CH, H), jnp.float32),
          pltpu.VMEM((CH, H), jnp.float32),
          pltpu.VMEM((CH, H), jnp.float32),
          pltpu.SemaphoreType.DMA,
          pltpu.SemaphoreType.DMA,
          pltpu.SemaphoreType.DMA,
      ],
      name=name,
  )
  return f(hin, fused, ep)


def _ln(h, g, b):
  mu = jnp.mean(h, axis=-1, keepdims=True)
  var = jnp.mean(jnp.square(h - mu), axis=-1, keepdims=True)
  return (h - mu) / jnp.sqrt(var + 1e-5) * g + b


def _tc_update_body(has_res, h2_ref, agg2_ref, W_ref, b_ref, g_ref,
                    beta_ref, *rest):
  if has_res:
    res_ref, h_out_ref, x2_out_ref = rest
  else:
    h_out_ref, x2_out_ref = rest
  a2 = agg2_ref[...]
  agg = jnp.concatenate([a2[:NN, :HH], a2[NN:, :HH]], axis=-1)
  t = (h2_ref[...] + agg) @ W_ref[...] + b_ref[...]
  if has_res:
    t = t + res_ref[...]
  h_out_ref[...] = t
  x2_out_ref[...] = jax.nn.relu(_ln(t, g_ref[...], beta_ref[...]))


def _tc_update(h2, agg2, W, b, g, beta, res=None):
  args = [h2, agg2, W, b, g, beta]
  if res is not None:
    args.append(res)
  return pl.pallas_call(
      functools.partial(_tc_update_body, res is not None),
      out_shape=[
          jax.ShapeDtypeStruct((NN, H), jnp.float32),
          jax.ShapeDtypeStruct((NN, H), jnp.float32),
      ],
  )(*args)


def _tc_final_body(h2_ref, agg2_ref, W_ref, b_ref, res_ref, g_ref,
                   beta_ref, batch_ref, out_ref):
  a2 = agg2_ref[...]
  agg = jnp.concatenate([a2[:NN, :HH], a2[NN:, :HH]], axis=-1)
  t = (h2_ref[...] + agg) @ W_ref[...] + b_ref[...] + res_ref[...]
  hn = _ln(t, g_ref[...], beta_ref[...])[:N]
  onehot = (batch_ref[...] == lax.broadcasted_iota(jnp.int32, (N, G), 1)
            ).astype(jnp.float32)
  sums = lax.dot_general(onehot, hn, (((0,), (0,)), ((), ())),
                         preferred_element_type=jnp.float32)
  counts = lax.dot_general(onehot, jnp.ones((N, 1), jnp.float32),
                           (((0,), (0,)), ((), ())),
                           preferred_element_type=jnp.float32)
  out_ref[...] = sums / jnp.maximum(counts, 1.0)


def _tc_final(h2, agg2, W, b, res, g, beta, batch2d):
  return pl.pallas_call(
      _tc_final_body,
      out_shape=jax.ShapeDtypeStruct((G, H), jnp.float32),
  )(h2, agg2, W, b, res, g, beta, batch2d)


def kernel(x, edge_index, edge_attr, batch, atom_tables, bond_tables,
           gcn_W, gcn_b, norm_gamma, norm_beta):
  i32 = jnp.int32
  xT = jnp.transpose(x).astype(i32)
  eaT = jnp.transpose(edge_attr).astype(i32)
  ei = edge_index.astype(i32)
  batch2d = batch.astype(i32).reshape(N, 1)
  W = [gcn_W[i] for i in range(3)]
  b = [gcn_b[i].reshape(1, H) for i in range(3)]
  g = [norm_gamma[i].reshape(1, H) for i in range(3)]
  beta = [norm_beta[i].reshape(1, H) for i in range(3)]

  h0, fused, ep = _tc_prep(xT, ei, eaT, atom_tables, bond_tables)

  agg0 = _sc_aggregate("sc_agg_0", h0, fused, ep)
  h_a, xc2 = _tc_update(h0, agg0, W[0], b[0], g[0], beta[0])
  agg1 = _sc_aggregate("sc_agg_1", xc2, fused, ep)
  h_b, xc3 = _tc_update(xc2, agg1, W[1], b[1], g[1], beta[1], res=h_a)
  agg2 = _sc_aggregate("sc_agg_2", xc3, fused, ep)
  return _tc_final(xc3, agg2, W[2], b[2], h_b, g[2], beta[2], batch2d)
